# Initial kernel scaffold; baseline (speedup 1.0000x reference)
import jax
import jax.numpy as jnp
from jax.experimental import pallas as pl


def kernel(node_embedding, batch, Wout, bout):
    raise NotImplementedError("write your pallas kernel here")


# speedup vs baseline: 1.4368x; 1.4368x over previous
import jax
import jax.numpy as jnp
from jax import lax
from jax.experimental import pallas as pl
from jax.experimental.pallas import tpu as pltpu
from jax.experimental.pallas import tpu_sc as plsc

_N = 100000
_D = 128
_S = 1024
_BLK = 2048
_NPAD = 100352
_TILES = 16
_CH = _NPAD // _TILES
_ROWS = _CH // 128


def _matvec_body(x_ref, w_ref, b_ref, o_ref):
    i = pl.program_id(0)
    y = jnp.dot(x_ref[...], w_ref[...], preferred_element_type=jnp.float32)
    y = y + b_ref[0, 0]
    row = i * _BLK + lax.broadcasted_iota(jnp.int32, (_BLK, 1), 0)
    o_ref[...] = jnp.where(row < _N, y, 0.0)


def _segment_body(y_hbm, idx_hbm, out_hbm, yv, iv, zv, acc):
    c = lax.axis_index("c")
    s = lax.axis_index("s")

    @pl.when(jnp.logical_and(c == 0, s == 0))
    def _zero():
        for i in range(_S // 16):
            zv[pl.ds(i * 16, 16)] = jnp.zeros((16,), jnp.float32)
        pltpu.sync_copy(zv, acc)

    plsc.subcore_barrier()

    @pl.when(c == 0)
    def _scatter():
        pltpu.sync_copy(idx_hbm.at[s], iv)
        pltpu.sync_copy(y_hbm.at[pl.ds(s * _CH, _CH)], yv)

        def body(j, carry):
            off = pl.multiple_of(j * 128, 128)
            pltpu.sync_copy(yv.at[pl.ds(off, 128)], acc.at[iv.at[j]], add=True)
            return carry

        lax.fori_loop(0, _ROWS, body, 0)

    plsc.subcore_barrier()

    @pl.when(jnp.logical_and(c == 0, s == 0))
    def _writeback():
        pltpu.sync_copy(acc, out_hbm)


def kernel(node_embedding, batch, Wout, bout):
    w = Wout.astype(jnp.float32)
    b2 = bout.reshape(1, 1).astype(jnp.float32)
    y = pl.pallas_call(
        _matvec_body,
        grid=(_NPAD // _BLK,),
        in_specs=[
            pl.BlockSpec((_BLK, _D), lambda i: (i, 0)),
            pl.BlockSpec((_D, 1), lambda i: (0, 0)),
            pl.BlockSpec((1, 1), lambda i: (0, 0), memory_space=pltpu.SMEM),
        ],
        out_specs=pl.BlockSpec((_BLK, 1), lambda i: (i, 0)),
        out_shape=jax.ShapeDtypeStruct((_NPAD, 1), jnp.float32),
    )(node_embedding, w, b2)
    y1 = y.reshape(_NPAD)
    idx = jnp.pad(batch.astype(jnp.int32), (0, _NPAD - _N)).reshape(_TILES, _ROWS, 128)

    seg = pl.kernel(
        _segment_body,
        out_type=jax.ShapeDtypeStruct((_S,), jnp.float32),
        mesh=plsc.VectorSubcoreMesh(core_axis_name="c", subcore_axis_name="s"),
        scratch_types=[
            pltpu.VMEM((_CH,), jnp.float32),
            pltpu.VMEM((_ROWS, 128), jnp.int32),
            pltpu.VMEM((_S,), jnp.float32),
            pltpu.VMEM_SHARED((_S,), jnp.float32),
        ],
    )(y1, idx)
    return seg


# speedup vs baseline: 2.0409x; 1.4205x over previous
import jax
import jax.numpy as jnp
from jax import lax
from jax.experimental import pallas as pl
from jax.experimental.pallas import tpu as pltpu
from jax.experimental.pallas import tpu_sc as plsc

_N = 100000
_D = 128
_S = 1024
_BLK = 2048
_NPAD = 100352
_NTILES = 14
_ROWS = 56
_ACC = 1152


def _matvec_body(x_ref, w_ref, b_ref, o_ref):
    y = jnp.dot(x_ref[...], w_ref[...], preferred_element_type=jnp.float32)
    o_ref[...] = (y + b_ref[0, 0]).reshape(_BLK // 128, 128)


def _segment_body(y_hbm, idx_hbm, out_hbm, yv, iv, zv, acc):
    c = lax.axis_index("c")
    s = lax.axis_index("s")

    @pl.when(jnp.logical_and(c == 0, s == 0))
    def _zero():
        for i in range(_ACC // 16):
            zv[pl.ds(i * 16, 16)] = jnp.zeros((16,), jnp.float32)
        pltpu.sync_copy(zv, acc)

    plsc.subcore_barrier()

    @pl.when(jnp.logical_and(c == 0, s < _NTILES))
    def _scatter():
        pltpu.sync_copy(idx_hbm.at[pl.ds(s * _ROWS, _ROWS)], iv)
        pltpu.sync_copy(y_hbm.at[pl.ds(s * _ROWS, _ROWS)], yv)

        def body(j, carry):
            pltpu.sync_copy(yv.at[j], acc.at[iv.at[j]], add=True)
            return carry

        lax.fori_loop(0, _ROWS, body, 0)

    plsc.subcore_barrier()

    @pl.when(jnp.logical_and(c == 0, s == 0))
    def _writeback():
        pltpu.sync_copy(acc.at[pl.ds(0, _S)], out_hbm)


def kernel(node_embedding, batch, Wout, bout):
    w = Wout.astype(jnp.float32)
    b2 = bout.reshape(1, 1).astype(jnp.float32)
    y = pl.pallas_call(
        _matvec_body,
        grid=(_NPAD // _BLK,),
        in_specs=[
            pl.BlockSpec((_BLK, _D), lambda i: (i, 0)),
            pl.BlockSpec((_D, 1), lambda i: (0, 0)),
            pl.BlockSpec((1, 1), lambda i: (0, 0), memory_space=pltpu.SMEM),
        ],
        out_specs=pl.BlockSpec((_BLK // 128, 128), lambda i: (i, 0)),
        out_shape=jax.ShapeDtypeStruct((_NPAD // 128, _D), jnp.float32),
    )(node_embedding, w, b2)

    dump = _S + (jnp.arange(_NPAD - _N, dtype=jnp.int32) % (_ACC - _S))
    idx = jnp.concatenate([batch.astype(jnp.int32), dump]).reshape(_NPAD // 128, 128)

    seg = pl.kernel(
        _segment_body,
        out_type=jax.ShapeDtypeStruct((_S,), jnp.float32),
        mesh=plsc.VectorSubcoreMesh(core_axis_name="c", subcore_axis_name="s"),
        scratch_types=[
            pltpu.VMEM((_ROWS, 128), jnp.float32),
            pltpu.VMEM((_ROWS, 128), jnp.int32),
            pltpu.VMEM((_ACC,), jnp.float32),
            pltpu.VMEM_SHARED((_ACC,), jnp.float32),
        ],
    )(y, idx)
    return seg


# speedup vs baseline: 3.3342x; 1.6337x over previous
import jax
import jax.numpy as jnp
from jax import lax
from jax.experimental import pallas as pl
from jax.experimental.pallas import tpu as pltpu
from jax.experimental.pallas import tpu_sc as plsc

_N = 100000
_D = 128
_S = 1024
_BLK = 2048
_NPAD = 100352
_NTILES = 14
_ROWS = 56
_ACC = 1152


_NSTREAM = 8
_TBLK = 1792
_TROWS = _TBLK // 128


def _matvec_body(x0, x1, x2, x3, x4, x5, x6, x7, w_ref, b_ref, o_ref):
    b = b_ref[0, 0]
    for k, xr in enumerate((x0, x1, x2, x3, x4, x5, x6, x7)):
        yt = lax.dot_general(
            w_ref[...], xr[...], (((1,), (1,)), ((), ())),
            preferred_element_type=jnp.float32,
        )
        o_ref[pl.ds(k * _TROWS, _TROWS), :] = yt.reshape(_TROWS, 128) + b


def _segment_body(y_hbm, idx_hbm, out_hbm, yv, iv, zv, acc):
    c = lax.axis_index("c")
    s = lax.axis_index("s")

    @pl.when(jnp.logical_and(c == 0, s == 0))
    def _zero():
        for i in range(_ACC // 16):
            zv[pl.ds(i * 16, 16)] = jnp.zeros((16,), jnp.float32)
        pltpu.sync_copy(zv, acc)

    plsc.subcore_barrier()

    @pl.when(jnp.logical_and(c == 0, s < _NTILES))
    def _scatter():
        pltpu.sync_copy(idx_hbm.at[pl.ds(s * _ROWS, _ROWS)], iv)
        pltpu.sync_copy(y_hbm.at[pl.ds(s * _ROWS, _ROWS)], yv)

        def body(j, carry):
            pltpu.sync_copy(yv.at[j], acc.at[iv.at[j]], add=True)
            return carry

        lax.fori_loop(0, _ROWS, body, 0)

    plsc.subcore_barrier()

    @pl.when(jnp.logical_and(c == 0, s == 0))
    def _writeback():
        pltpu.sync_copy(acc.at[pl.ds(0, _S)], out_hbm)


def kernel(node_embedding, batch, Wout, bout):
    w = Wout.astype(jnp.float32).reshape(1, _D)
    b2 = bout.reshape(1, 1).astype(jnp.float32)
    y = pl.pallas_call(
        _matvec_body,
        grid=(_NPAD // (_NSTREAM * _TBLK),),
        in_specs=[
            pl.BlockSpec((_TBLK, _D), lambda i, k=k: (_NSTREAM * i + k, 0))
            for k in range(_NSTREAM)
        ] + [
            pl.BlockSpec((1, _D), lambda i: (0, 0)),
            pl.BlockSpec((1, 1), lambda i: (0, 0), memory_space=pltpu.SMEM),
        ],
        out_specs=pl.BlockSpec((_NSTREAM * _TROWS, 128), lambda i: (i, 0)),
        out_shape=jax.ShapeDtypeStruct((_NPAD // 128, _D), jnp.float32),
    )(*([node_embedding] * _NSTREAM), w, b2)

    dump = _S + (jnp.arange(_NPAD - _N, dtype=jnp.int32) % (_ACC - _S))
    idx = jnp.concatenate([batch.astype(jnp.int32), dump]).reshape(_NPAD // 128, 128)

    seg = pl.kernel(
        _segment_body,
        out_type=jax.ShapeDtypeStruct((_S,), jnp.float32),
        mesh=plsc.VectorSubcoreMesh(core_axis_name="c", subcore_axis_name="s"),
        scratch_types=[
            pltpu.VMEM((_ROWS, 128), jnp.float32),
            pltpu.VMEM((_ROWS, 128), jnp.int32),
            pltpu.VMEM((_ACC,), jnp.float32),
            pltpu.VMEM_SHARED((_ACC,), jnp.float32),
        ],
    )(y, idx)
    return seg


# speedup vs baseline: 3.3363x; 1.0006x over previous
import jax
import jax.numpy as jnp
from jax import lax
from jax.experimental import pallas as pl
from jax.experimental.pallas import tpu as pltpu
from jax.experimental.pallas import tpu_sc as plsc

_N = 100000
_D = 128
_S = 1024
_BLK = 2048
_NPAD = 100352
_NTILES = 14
_ROWS = 56
_CHUNK = _ROWS * 128
_ACC = 1152


_NSTREAM = 8
_TBLK = 1792
_TROWS = _TBLK // 128


def _matvec_body(x0, x1, x2, x3, x4, x5, x6, x7, w_ref, b_ref, o_ref):
    b = b_ref[0, 0]
    for k, xr in enumerate((x0, x1, x2, x3, x4, x5, x6, x7)):
        yt = lax.dot_general(
            w_ref[...], xr[...], (((1,), (1,)), ((), ())),
            preferred_element_type=jnp.float32,
        )
        o_ref[pl.ds(k * _TROWS, _TROWS), :] = yt.reshape(_TROWS, 128) + b


def _segment_body(y_hbm, idx_hbm, out_hbm, yv, iv, zv, acc):
    c = lax.axis_index("c")
    s = lax.axis_index("s")

    @pl.when(jnp.logical_and(c == 0, s == 0))
    def _zero():
        for i in range(_ACC // 16):
            zv[pl.ds(i * 16, 16)] = jnp.zeros((16,), jnp.float32)
        pltpu.sync_copy(zv, acc)

    plsc.subcore_barrier()

    @pl.when(jnp.logical_and(c == 0, s < _NTILES))
    def _scatter():
        pltpu.sync_copy(idx_hbm.at[pl.ds(s * _CHUNK, _CHUNK)], iv)
        pltpu.sync_copy(y_hbm.at[pl.ds(s * _CHUNK, _CHUNK)], yv)

        pltpu.sync_copy(yv, acc.at[iv], add=True)

    plsc.subcore_barrier()

    @pl.when(jnp.logical_and(c == 0, s == 0))
    def _writeback():
        pltpu.sync_copy(acc.at[pl.ds(0, _S)], out_hbm)


def kernel(node_embedding, batch, Wout, bout):
    w = Wout.astype(jnp.float32).reshape(1, _D)
    b2 = bout.reshape(1, 1).astype(jnp.float32)
    y = pl.pallas_call(
        _matvec_body,
        grid=(_NPAD // (_NSTREAM * _TBLK),),
        in_specs=[
            pl.BlockSpec((_TBLK, _D), lambda i, k=k: (_NSTREAM * i + k, 0))
            for k in range(_NSTREAM)
        ] + [
            pl.BlockSpec((1, _D), lambda i: (0, 0)),
            pl.BlockSpec((1, 1), lambda i: (0, 0), memory_space=pltpu.SMEM),
        ],
        out_specs=pl.BlockSpec((_NSTREAM * _TROWS, 128), lambda i: (i, 0)),
        out_shape=jax.ShapeDtypeStruct((_NPAD // 128, _D), jnp.float32),
    )(*([node_embedding] * _NSTREAM), w, b2)

    dump = _S + (jnp.arange(_NPAD - _N, dtype=jnp.int32) % (_ACC - _S))
    idx = jnp.concatenate([batch.astype(jnp.int32), dump])

    seg = pl.kernel(
        _segment_body,
        out_type=jax.ShapeDtypeStruct((_S,), jnp.float32),
        mesh=plsc.VectorSubcoreMesh(core_axis_name="c", subcore_axis_name="s"),
        scratch_types=[
            pltpu.VMEM((_CHUNK,), jnp.float32),
            pltpu.VMEM((_CHUNK,), jnp.int32),
            pltpu.VMEM((_ACC,), jnp.float32),
            pltpu.VMEM_SHARED((_ACC,), jnp.float32),
        ],
    )(y.reshape(_NPAD), idx)
    return seg
